# Initial kernel scaffold; baseline (speedup 1.0000x reference)
import jax
import jax.numpy as jnp
from jax.experimental import pallas as pl


def kernel(hidden_states, attn_weights, keys):
    raise NotImplementedError("write your pallas kernel here")


# speedup vs baseline: 1.0323x; 1.0323x over previous
import math

import jax
import jax.numpy as jnp
from jax.experimental import pallas as pl
from jax.experimental.pallas import tpu as pltpu

TAU_FEAT = 0.2
TAU_SIM = 0.1
EPS = 1e-12
A_ATTN, A_ENT, A_MUT = 1.0, 0.4, 0.6
K_DOM = 64
CTX_NUM = 16
N = 576
D = 768
NR = N - K_DOM
STEP = NR // CTX_NUM

DEFAULT = jax.lax.Precision.DEFAULT
HIGHEST = jax.lax.Precision.HIGHEST


def _normalize_vz(x, eps=EPS):
    n = jnp.linalg.norm(x, axis=-1, keepdims=True)
    return x / jnp.maximum(n, eps)


def _minmax_vz(s):
    lo = s.min(axis=1, keepdims=True)
    hi = s.max(axis=1, keepdims=True)
    return (s - lo) / (hi - lo + EPS)


def _hybrid_scores_vz(attn_weights, keys):
    s_attn = attn_weights[:, :, 0, 1:].mean(axis=1)
    x = keys[:, 1:, :].astype(jnp.float32)
    z = _normalize_vz(x)
    p = jax.nn.softmax(z / TAU_FEAT, axis=-1)
    Hent = -(p * jnp.log(p + EPS)).sum(axis=-1) / math.log(x.shape[-1] + EPS)
    sim = jnp.einsum('bnd,bmd->bnm', z, z)
    eye = jnp.eye(sim.shape[-1], dtype=bool)[None]
    sim = jnp.where(eye, -1e9, sim)
    q = jax.nn.softmax(sim / TAU_SIM, axis=-1)
    Hsim = -(q * jnp.log(q + EPS)).sum(axis=-1) / math.log(q.shape[-1] + EPS)
    I = 1.0 - Hsim
    return A_ATTN * _minmax_vz(s_attn) + A_ENT * _minmax_vz(Hent) + A_MUT * _minmax_vz(I)


def _vz_kernel(dom_ref, rem_ref, h_ref, k_ref, out_ref):
    h = h_ref[0]
    kx = k_ref[0]
    dom_idx = dom_ref[0]
    rem_idx = rem_ref[0]

    row_n = jax.lax.broadcasted_iota(jnp.int32, (N, K_DOM), 0)
    oh_dom = (row_n == dom_idx).astype(jnp.float32)
    dom = jax.lax.dot_general(oh_dom, h, (((0,), (0,)), ((), ())),
                              precision=HIGHEST,
                              preferred_element_type=jnp.float32)

    row_r = jax.lax.broadcasted_iota(jnp.int32, (N, NR), 0)
    oh_rem = (row_r == rem_idx).astype(jnp.float32)
    h_rem = jax.lax.dot_general(oh_rem, h, (((0,), (0,)), ((), ())),
                                precision=HIGHEST,
                                preferred_element_type=jnp.float32)
    k_rem = jax.lax.dot_general(oh_rem, kx, (((0,), (0,)), ((), ())),
                                precision=HIGHEST,
                                preferred_element_type=jnp.float32)

    nrm = jnp.sqrt(jnp.sum(k_rem * k_rem, axis=-1, keepdims=True))
    z = k_rem / jnp.maximum(nrm, EPS)

    pos = jax.lax.broadcasted_iota(jnp.int32, (NR, CTX_NUM), 0)
    cc = jax.lax.broadcasted_iota(jnp.int32, (NR, CTX_NUM), 1)
    oh_tgt = (pos == cc * STEP).astype(jnp.float32)
    tgt_z = jax.lax.dot_general(oh_tgt, z, (((0,), (0,)), ((), ())),
                                precision=HIGHEST,
                                preferred_element_type=jnp.float32)
    tgt_h = jax.lax.dot_general(oh_tgt, h_rem, (((0,), (0,)), ((), ())),
                                precision=HIGHEST,
                                preferred_element_type=jnp.float32)

    s_tgt = jax.lax.dot_general(z, tgt_z, (((1,), (1,)), ((), ())),
                                precision=DEFAULT,
                                preferred_element_type=jnp.float32)
    maxv = jnp.max(s_tgt, axis=1, keepdims=True)
    first_c = jnp.min(jnp.where(s_tgt == maxv, cc, CTX_NUM), axis=1)
    pos1 = jax.lax.broadcasted_iota(jnp.int32, (NR, CTX_NUM), 0)
    is_tgt_row = (jnp.remainder(pos1[:, 0], STEP) == 0)
    oh_asg = ((cc == first_c[:, None]) & (~is_tgt_row)[:, None]).astype(jnp.float32)
    counts = jnp.sum(oh_asg, axis=0)
    sums = jax.lax.dot_general(oh_asg, h_rem, (((0,), (0,)), ((), ())),
                               precision=DEFAULT,
                               preferred_element_type=jnp.float32)
    means = sums / jnp.maximum(counts[:, None], 1.0)
    ctx = jnp.where(counts[:, None] > 0, means, tgt_h)

    out_ref[0, 0:K_DOM, :] = dom
    out_ref[0, K_DOM:K_DOM + CTX_NUM, :] = ctx


def kernel(hidden_states, attn_weights, keys):
    b = hidden_states.shape[0]
    scores = _hybrid_scores_vz(attn_weights, keys)
    patches = hidden_states[:, 1:, :]
    keys_p = keys[:, 1:, :]
    _, dom_idx = jax.lax.top_k(scores, K_DOM)
    order = jnp.argsort(-scores, axis=1)
    rem_idx = jnp.sort(order[:, K_DOM:], axis=1)

    out = pl.pallas_call(
        _vz_kernel,
        grid=(b,),
        in_specs=[
            pl.BlockSpec((1, 1, K_DOM), lambda i: (i, 0, 0)),
            pl.BlockSpec((1, 1, NR), lambda i: (i, 0, 0)),
            pl.BlockSpec((1, N, D), lambda i: (i, 0, 0)),
            pl.BlockSpec((1, N, D), lambda i: (i, 0, 0)),
        ],
        out_specs=pl.BlockSpec((1, K_DOM + CTX_NUM, D), lambda i: (i, 0, 0)),
        out_shape=jax.ShapeDtypeStruct((b, K_DOM + CTX_NUM, D), jnp.float32),
        compiler_params=pltpu.CompilerParams(
            dimension_semantics=("parallel",)),
    )(dom_idx[:, None, :], rem_idx[:, None, :], patches, keys_p)

    cls = hidden_states[:, :1, :]
    return jnp.concatenate([cls, out], axis=1)
